# baseline (speedup 1.0000x reference)
import functools

import jax
import jax.numpy as jnp
from jax import lax
from jax.experimental import pallas as pl
from jax.experimental.pallas import tpu as pltpu
from jax.experimental.pallas import tpu_sc as plsc

N_ENTITY = 1000000
N_RELATION = 1000
BATCH = 16384
DIM = 64

NC = 2
NS = 16
L = 16
NW = NC * NS
B_PER_W = BATCH // NW
CHUNK = 256
IDX_CHUNK = 128


def _body(head_hbm, tail_hbm, rel_hbm, emb_e2_hbm, emb_r2_hbm, out_hbm,
          hidx_v, tidx_v, ridx_v, hrow_v, trow_v, rrow_v,
          h2_v, t2_v, r2_v, out_v, sem):
    wid = lax.axis_index("s") * NC + lax.axis_index("c")
    base = wid * B_PER_W
    lane = lax.broadcasted_iota(jnp.int32, (L,), 0)

    for half in range(B_PER_W // CHUNK):
        b0 = base + half * CHUNK
        pltpu.sync_copy(head_hbm.at[pl.ds(b0, CHUNK)], hidx_v)
        pltpu.sync_copy(tail_hbm.at[pl.ds(b0, CHUNK)], tidx_v)
        pltpu.sync_copy(rel_hbm.at[pl.ds(b0, CHUNK)], ridx_v)

        def rows(j, carry):
            sl = pl.ds(j * L, L)
            hrow_v[sl] = jax.lax.shift_right_logical(hidx_v[sl], 1)
            trow_v[sl] = jax.lax.shift_right_logical(tidx_v[sl], 1)
            rrow_v[sl] = jax.lax.shift_right_logical(ridx_v[sl], 1)
            return carry

        lax.fori_loop(0, CHUNK // L, rows, 0)

        copies = []
        for j in range(CHUNK // IDX_CHUNK):
            sl = pl.ds(j * IDX_CHUNK, IDX_CHUNK)
            copies.append(pltpu.async_copy(emb_e2_hbm.at[hrow_v.at[sl]], h2_v.at[sl], sem))
            copies.append(pltpu.async_copy(emb_e2_hbm.at[trow_v.at[sl]], t2_v.at[sl], sem))
            copies.append(pltpu.async_copy(emb_r2_hbm.at[rrow_v.at[sl]], r2_v.at[sl], sem))
        for c in copies:
            c.wait()

        def group(g, carry):
            sl = pl.ds(g * L, L)
            rvec = g * L + lane
            hoff = jax.lax.shift_left(hidx_v[sl] & 1, 6)
            toff = jax.lax.shift_left(tidx_v[sl] & 1, 6)
            roff = jax.lax.shift_left(ridx_v[sl] & 1, 6)
            acc = jnp.zeros((L,), jnp.float32)
            for d in range(DIM):
                hh = plsc.load_gather(h2_v, [rvec, hoff + d])
                tt = plsc.load_gather(t2_v, [rvec, toff + d])
                rr = plsc.load_gather(r2_v, [rvec, roff + d])
                acc = acc + hh * tt * rr
            out_v[pl.ds(half * CHUNK + g * L, L)] = acc
            return carry

        lax.fori_loop(0, CHUNK // L, group, 0)

    pltpu.sync_copy(out_v, out_hbm.at[pl.ds(base, B_PER_W)])


@jax.jit
def kernel(head, tail, relation, emb_E, emb_R):
    mesh = plsc.VectorSubcoreMesh(
        core_axis_name="c", subcore_axis_name="s", num_cores=NC, num_subcores=NS
    )
    run = pl.kernel(
        _body,
        out_type=jax.ShapeDtypeStruct((BATCH,), jnp.float32),
        mesh=mesh,
        compiler_params=pltpu.CompilerParams(
            needs_layout_passes=False, use_tc_tiling_on_sc=True
        ),
        scratch_types=[
            pltpu.VMEM((CHUNK,), jnp.int32),
            pltpu.VMEM((CHUNK,), jnp.int32),
            pltpu.VMEM((CHUNK,), jnp.int32),
            pltpu.VMEM((CHUNK,), jnp.int32),
            pltpu.VMEM((CHUNK,), jnp.int32),
            pltpu.VMEM((CHUNK,), jnp.int32),
            pltpu.VMEM((CHUNK, 128), jnp.float32),
            pltpu.VMEM((CHUNK, 128), jnp.float32),
            pltpu.VMEM((CHUNK, 128), jnp.float32),
            pltpu.VMEM((B_PER_W,), jnp.float32),
            pltpu.SemaphoreType.DMA,
        ],
    )
    return run(head.astype(jnp.int32), tail.astype(jnp.int32),
               relation.astype(jnp.int32),
               emb_E.reshape(N_ENTITY // 2, 128),
               emb_R.reshape(N_RELATION // 2, 128))


# speedup vs baseline: 1.3323x; 1.3323x over previous
import functools

import jax
import jax.numpy as jnp
from jax import lax
from jax.experimental import pallas as pl
from jax.experimental.pallas import tpu as pltpu
from jax.experimental.pallas import tpu_sc as plsc

N_ENTITY = 1000000
N_RELATION = 1000
BATCH = 16384
DIM = 64

NC = 2
NS = 16
L = 16
NW = NC * NS
B_PER_W = BATCH // NW
CHUNK = 32


def _body(head_hbm, tail_hbm, rel_hbm, emb_e_hbm, emb_r_hbm, out_hbm,
          hidx_v, tidx_v, ridx_v,
          hb_v, tb_v, rb_v, out_v, sem):
    wid = lax.axis_index("s") * NC + lax.axis_index("c")
    base = wid * B_PER_W
    lane = lax.broadcasted_iota(jnp.int32, (L,), 0)

    for half in range(B_PER_W // CHUNK):
        b0 = base + half * CHUNK
        pltpu.sync_copy(head_hbm.at[pl.ds(b0, CHUNK)], hidx_v)
        pltpu.sync_copy(tail_hbm.at[pl.ds(b0, CHUNK)], tidx_v)
        pltpu.sync_copy(rel_hbm.at[pl.ds(b0, CHUNK)], ridx_v)
        def issue(g, carry):
            sl = pl.ds(g * L, L)
            hv = hidx_v[sl]
            tv = tidx_v[sl]
            rv = ridx_v[sl]
            for k in range(L):
                eh = pl.multiple_of((hv[k] >> 3) * 8, 8)
                et = pl.multiple_of((tv[k] >> 3) * 8, 8)
                er = pl.multiple_of((rv[k] >> 3) * 8, 8)
                dst = pl.ds(pl.multiple_of(g * (L * 8), 8) + k * 8, 8)
                pltpu.async_copy(emb_e_hbm.at[pl.ds(eh, 8), :], hb_v.at[dst], sem)
                pltpu.async_copy(emb_e_hbm.at[pl.ds(et, 8), :], tb_v.at[dst], sem)
                pltpu.async_copy(emb_r_hbm.at[pl.ds(er, 8), :], rb_v.at[dst], sem)
            return carry

        lax.fori_loop(0, CHUNK // L, issue, 0)

        pltpu.make_async_copy(emb_e_hbm.at[pl.ds(0, CHUNK * 8), :], hb_v, sem).wait()
        pltpu.make_async_copy(emb_e_hbm.at[pl.ds(0, CHUNK * 8), :], tb_v, sem).wait()
        pltpu.make_async_copy(emb_e_hbm.at[pl.ds(0, CHUNK * 8), :], rb_v, sem).wait()

        def group(g, carry):
            sl = pl.ds(g * L, L)
            blk = (g * L + lane) * 8
            hrow = blk + (hidx_v[sl] & 7)
            trow = blk + (tidx_v[sl] & 7)
            rrow = blk + (ridx_v[sl] & 7)
            acc = jnp.zeros((L,), jnp.float32)
            for d in range(DIM):
                hh = plsc.load_gather(hb_v, [hrow, jnp.full((L,), d, jnp.int32)])
                tt = plsc.load_gather(tb_v, [trow, jnp.full((L,), d, jnp.int32)])
                rr = plsc.load_gather(rb_v, [rrow, jnp.full((L,), d, jnp.int32)])
                acc = acc + hh * tt * rr
            out_v[pl.ds(half * CHUNK + g * L, L)] = acc
            return carry

        lax.fori_loop(0, CHUNK // L, group, 0)

    pltpu.sync_copy(out_v, out_hbm.at[pl.ds(base, B_PER_W)])


@jax.jit
def kernel(head, tail, relation, emb_E, emb_R):
    mesh = plsc.VectorSubcoreMesh(
        core_axis_name="c", subcore_axis_name="s", num_cores=NC, num_subcores=NS
    )
    run = pl.kernel(
        _body,
        out_type=jax.ShapeDtypeStruct((BATCH,), jnp.float32),
        mesh=mesh,
        compiler_params=pltpu.CompilerParams(
            needs_layout_passes=False, use_tc_tiling_on_sc=True
        ),
        scratch_types=[
            pltpu.VMEM((CHUNK,), jnp.int32),
            pltpu.VMEM((CHUNK,), jnp.int32),
            pltpu.VMEM((CHUNK,), jnp.int32),
            pltpu.VMEM((CHUNK * 8, DIM), jnp.float32),
            pltpu.VMEM((CHUNK * 8, DIM), jnp.float32),
            pltpu.VMEM((CHUNK * 8, DIM), jnp.float32),
            pltpu.VMEM((B_PER_W,), jnp.float32),
            pltpu.SemaphoreType.DMA,
        ],
    )
    return run(head.astype(jnp.int32), tail.astype(jnp.int32),
               relation.astype(jnp.int32), emb_E, emb_R)


# speedup vs baseline: 1.9969x; 1.4989x over previous
import functools

import jax
import jax.numpy as jnp
from jax import lax
from jax.experimental import pallas as pl
from jax.experimental.pallas import tpu as pltpu
from jax.experimental.pallas import tpu_sc as plsc

N_ENTITY = 1000000
N_RELATION = 1000
BATCH = 16384
DIM = 64

NC = 2
NS = 16
L = 16
NW = NC * NS
PANEL = 128
N_PANEL_FULL = N_ENTITY // PANEL
N_PANELS = N_PANEL_FULL + 1
P_PER_W = (N_PANELS + NW - 1) // NW
RING = 4
MAXM = 2 * BATCH + L
B_PER_W = BATCH // NW
K2CHUNK = 128

_params = pltpu.CompilerParams(needs_layout_passes=False, use_tc_tiling_on_sc=True)
_mesh = dict(core_axis_name="c", subcore_axis_name="s", num_cores=NC, num_subcores=NS)


def _sweep_body(head_hbm, tail_hbm, emb_et_hbm, tail_panel_hbm, hm_hbm,
                idx_v, ents_v, bents_v, panels_v, orows_v,
                cnt_s, hist_s, start_s,
                sems, osems, sem):
    wid = lax.axis_index("s") * NC + lax.axis_index("c")
    c0 = wid * P_PER_W
    c1 = jnp.minimum(c0 + P_PER_W, N_PANELS)
    e_lo = c0 * PANEL
    e_hi = jnp.minimum(c1 * PANEL, N_ENTITY)
    lane = lax.broadcasted_iota(jnp.int32, (L,), 0)

    cnt_s[0] = 0

    def scan_block(role, src_hbm):
        def blk(i, carry):
            pltpu.sync_copy(src_hbm.at[pl.ds(i * 2048, 2048)], idx_v)

            def chunk(j, carry2):
                e = idx_v[pl.ds(j * L, L)]
                m = (e >= e_lo) & (e < e_hi)
                npop = plsc.all_reduce_population_count(m)
                cnt = cnt_s[0]
                slot = (i * 2048 + j * L + role * BATCH) + lane
                packed = (jax.lax.shift_left((e >> 7) - c0, 22)
                          | jax.lax.shift_left(slot, 7) | (e & 127))
                plsc.store_compressed(ents_v.at[pl.ds(cnt, L)], packed, mask=m)
                cnt_s[0] = cnt + npop[0]
                return carry2

            return lax.fori_loop(0, 2048 // L, chunk, carry)

        lax.fori_loop(0, BATCH // 2048, blk, 0)

    scan_block(0, head_hbm)
    scan_block(1, tail_hbm)
    nmatch = cnt_s[0]

    def hzero(p, carry):
        hist_s[p] = 0
        return carry

    lax.fori_loop(0, P_PER_W, hzero, 0)

    def hcount(i, carry):
        ev = ents_v[pl.ds(i * L, L)]
        for k in range(L):
            @pl.when((i * L + k) < nmatch)
            def _():
                p = ev[k] >> 22
                hist_s[p] = hist_s[p] + 1
        return carry

    lax.fori_loop(0, (nmatch + L - 1) // L, hcount, 0)

    def prefix(p, acc):
        h = hist_s[p]
        start_s[p] = acc
        return acc + h

    lax.fori_loop(0, P_PER_W, prefix, 0)

    def bsort(i, carry):
        ev = ents_v[pl.ds(i * L, L)]
        for k in range(L):
            @pl.when((i * L + k) < nmatch)
            def _():
                p = ev[k] >> 22
                pos = start_s[p]
                onek = lane == k
                plsc.store_compressed(bents_v.at[pl.ds(pos, L)],
                                      jnp.full((L,), ev[k], jnp.int32), mask=onek)
                start_s[p] = pos + 1
        return carry

    lax.fori_loop(0, (nmatch + L - 1) // L, bsort, 0)

    n_my_panels = c1 - c0

    def fire(slot, ci):
        @pl.when(ci < n_my_panels)
        def _():
            c = c0 + ci

            @pl.when(c < N_PANEL_FULL)
            def _():
                off = pl.multiple_of(c * PANEL, PANEL)
                pltpu.async_copy(emb_et_hbm.at[:, pl.ds(off, PANEL)],
                                 panels_v.at[slot], sems[slot])

            @pl.when(c >= N_PANEL_FULL)
            def _():
                pltpu.async_copy(tail_panel_hbm, panels_v.at[slot], sems[slot])

    for slot in range(RING):
        fire(slot, jnp.int32(slot))

    def wave(v, carry):
        for slot in range(RING):
            ci = v * RING + slot

            @pl.when(ci < n_my_panels)
            def _():
                c = c0 + ci
                pltpu.make_async_copy(tail_panel_hbm, panels_v.at[slot],
                                      sems[slot]).wait()
                s = jnp.where(c > c0, start_s[jnp.maximum(ci - 1, 0)], 0)
                e = start_s[ci]

                def mchunk(i, carry2):
                    m0 = s + i * L
                    valid = (m0 + lane) < e
                    pk = bents_v[pl.ds(m0, L)]
                    pk = jnp.where(valid, pk, jnp.full((L,), pk[0], jnp.int32))
                    col = pk & 127
                    mv = (pk >> 7) & 32767
                    for d in range(DIM):
                        dsp = jnp.full((L,), d, jnp.int32)
                        vals = plsc.load_gather(panels_v.at[slot], [dsp, col])
                        plsc.store_scatter(orows_v.at[slot], [lane, dsp], vals)
                    pltpu.async_copy(orows_v.at[slot], hm_hbm.at[mv],
                                     osems[slot]).wait()
                    return carry2

                nchunks = jnp.maximum((e - s + L - 1) // L, 0)
                lax.fori_loop(0, nchunks, mchunk, 0)
                fire(slot, ci + RING)
        return carry

    lax.fori_loop(0, (P_PER_W + RING - 1) // RING, wave, 0)


def _score_body(rel_hbm, hm_hbm, emb_rt_hbm, out_hbm,
                ridx_v, h_v, t_v, rt_v, out_v, sem):
    wid = lax.axis_index("s") * NC + lax.axis_index("c")
    base = wid * B_PER_W
    lane = lax.broadcasted_iota(jnp.int32, (L,), 0)

    for p in range(8):
        off = pl.multiple_of(p * PANEL, PANEL)
        pltpu.async_copy(emb_rt_hbm.at[:, pl.ds(off, PANEL)],
                         rt_v.at[:, pl.ds(off, PANEL)], sem)
    pltpu.make_async_copy(emb_rt_hbm, rt_v, sem).wait()

    for half in range(B_PER_W // K2CHUNK):
        b0 = base + half * K2CHUNK
        pltpu.sync_copy(rel_hbm.at[pl.ds(b0, K2CHUNK)], ridx_v)
        pltpu.async_copy(hm_hbm.at[pl.ds(b0, K2CHUNK)], h_v, sem)
        pltpu.async_copy(hm_hbm.at[pl.ds(BATCH + b0, K2CHUNK)], t_v, sem)
        pltpu.make_async_copy(hm_hbm.at[pl.ds(0, K2CHUNK)], h_v, sem).wait()
        pltpu.make_async_copy(hm_hbm.at[pl.ds(0, K2CHUNK)], t_v, sem).wait()

        def group(g, carry):
            rows = g * L + lane
            rel = ridx_v[pl.ds(g * L, L)]
            acc = jnp.zeros((L,), jnp.float32)
            for d in range(DIM):
                dsp = jnp.full((L,), d, jnp.int32)
                hh = plsc.load_gather(h_v, [rows, dsp])
                tt = plsc.load_gather(t_v, [rows, dsp])
                rr = plsc.load_gather(rt_v, [dsp, rel])
                acc = acc + hh * tt * rr
            out_v[pl.ds(half * K2CHUNK + g * L, L)] = acc
            return carry

        lax.fori_loop(0, K2CHUNK // L, group, 0)

    pltpu.sync_copy(out_v, out_hbm.at[pl.ds(base, B_PER_W)])


@jax.jit
def kernel(head, tail, relation, emb_E, emb_R):
    head = head.astype(jnp.int32)
    tail = tail.astype(jnp.int32)
    relation = relation.astype(jnp.int32)
    emb_et = emb_E.T
    tail_panel = jnp.pad(emb_E[N_PANEL_FULL * PANEL:], ((0, 64), (0, 0))).T
    emb_rt = jnp.pad(emb_R, ((0, 1024 - N_RELATION), (0, 0))).T

    sweep = pl.kernel(
        _sweep_body,
        out_type=jax.ShapeDtypeStruct((2 * BATCH, PANEL), jnp.float32),
        mesh=plsc.VectorSubcoreMesh(**_mesh),
        compiler_params=_params,
        scratch_types=[
            pltpu.VMEM((2048,), jnp.int32),
            pltpu.VMEM((MAXM,), jnp.int32),
            pltpu.VMEM((MAXM,), jnp.int32),
            pltpu.VMEM((RING, DIM, PANEL), jnp.float32),
            pltpu.VMEM((RING, L, PANEL), jnp.float32),
            pltpu.SMEM((1,), jnp.int32),
            pltpu.SMEM((P_PER_W,), jnp.int32),
            pltpu.SMEM((P_PER_W,), jnp.int32),
            [pltpu.SemaphoreType.DMA] * RING,
            [pltpu.SemaphoreType.DMA] * RING,
            pltpu.SemaphoreType.DMA,
        ],
    )
    hm = sweep(head, tail, emb_et, tail_panel)

    score = pl.kernel(
        _score_body,
        out_type=jax.ShapeDtypeStruct((BATCH,), jnp.float32),
        mesh=plsc.VectorSubcoreMesh(**_mesh),
        compiler_params=_params,
        scratch_types=[
            pltpu.VMEM((K2CHUNK,), jnp.int32),
            pltpu.VMEM((K2CHUNK, PANEL), jnp.float32),
            pltpu.VMEM((K2CHUNK, PANEL), jnp.float32),
            pltpu.VMEM((DIM, 1024), jnp.float32),
            pltpu.VMEM((B_PER_W,), jnp.float32),
            pltpu.SemaphoreType.DMA,
        ],
    )
    return score(relation, hm, emb_rt)


# speedup vs baseline: 2.2097x; 1.1066x over previous
import functools

import jax
import jax.numpy as jnp
from jax import lax
from jax.experimental import pallas as pl
from jax.experimental.pallas import tpu as pltpu
from jax.experimental.pallas import tpu_sc as plsc

N_ENTITY = 1000000
N_RELATION = 1000
BATCH = 16384
DIM = 64

NC = 2
NS = 16
L = 16
NW = NC * NS
PANEL = 128
N_PANEL_FULL = N_ENTITY // PANEL
N_PANELS = N_PANEL_FULL + 1
P_PER_W = (N_PANELS + NW - 1) // NW
RING = 4
MAXM = 2 * BATCH + L
B_PER_W = BATCH // NW
K2CHUNK = 128

_params = pltpu.CompilerParams(needs_layout_passes=False, use_tc_tiling_on_sc=True)
_mesh = dict(core_axis_name="c", subcore_axis_name="s", num_cores=NC, num_subcores=NS)


def _sweep_body(head_hbm, tail_hbm, emb_et_hbm, tail_panel_hbm, hm_hbm,
                idx_v, ents_v, bents_v, panels_v, orows_v,
                cnt_s, hist_s, start_s, pend_s,
                sems, osems, sem):
    wid = lax.axis_index("s") * NC + lax.axis_index("c")
    c0 = wid * P_PER_W
    c1 = jnp.minimum(c0 + P_PER_W, N_PANELS)
    e_lo = c0 * PANEL
    e_hi = jnp.minimum(c1 * PANEL, N_ENTITY)
    lane = lax.broadcasted_iota(jnp.int32, (L,), 0)

    cnt_s[0] = 0
    for slot in range(RING):
        pend_s[slot] = 0

    def scan_block(role, src_hbm):
        def blk(i, carry):
            pltpu.sync_copy(src_hbm.at[pl.ds(i * 2048, 2048)], idx_v)

            def chunk(j, carry2):
                e = idx_v[pl.ds(j * L, L)]
                m = (e >= e_lo) & (e < e_hi)
                npop = plsc.all_reduce_population_count(m)
                cnt = cnt_s[0]
                slot = (i * 2048 + j * L + role * BATCH) + lane
                packed = (jax.lax.shift_left((e >> 7) - c0, 22)
                          | jax.lax.shift_left(slot, 7) | (e & 127))
                plsc.store_compressed(ents_v.at[pl.ds(cnt, L)], packed, mask=m)
                cnt_s[0] = cnt + npop[0]
                return carry2

            return lax.fori_loop(0, 2048 // L, chunk, carry)

        lax.fori_loop(0, BATCH // 2048, blk, 0)

    scan_block(0, head_hbm)
    scan_block(1, tail_hbm)
    nmatch = cnt_s[0]

    def hzero(p, carry):
        hist_s[p] = 0
        return carry

    lax.fori_loop(0, P_PER_W, hzero, 0)

    def hcount(i, carry):
        ev = ents_v[pl.ds(i * L, L)]
        for k in range(L):
            @pl.when((i * L + k) < nmatch)
            def _():
                p = ev[k] >> 22
                hist_s[p] = hist_s[p] + 1
        return carry

    lax.fori_loop(0, (nmatch + L - 1) // L, hcount, 0)

    def prefix(p, acc):
        h = hist_s[p]
        start_s[p] = acc
        return acc + h

    lax.fori_loop(0, P_PER_W, prefix, 0)

    def bsort(i, carry):
        ev = ents_v[pl.ds(i * L, L)]
        for k in range(L):
            @pl.when((i * L + k) < nmatch)
            def _():
                p = ev[k] >> 22
                pos = start_s[p]
                onek = lane == k
                plsc.store_compressed(bents_v.at[pl.ds(pos, L)],
                                      jnp.full((L,), ev[k], jnp.int32), mask=onek)
                start_s[p] = pos + 1
        return carry

    lax.fori_loop(0, (nmatch + L - 1) // L, bsort, 0)

    n_my_panels = c1 - c0

    def fire(slot, ci):
        @pl.when(ci < n_my_panels)
        def _():
            c = c0 + ci

            @pl.when(c < N_PANEL_FULL)
            def _():
                off = pl.multiple_of(c * PANEL, PANEL)
                pltpu.async_copy(emb_et_hbm.at[:, pl.ds(off, PANEL)],
                                 panels_v.at[slot], sems[slot])

            @pl.when(c >= N_PANEL_FULL)
            def _():
                pltpu.async_copy(tail_panel_hbm, panels_v.at[slot], sems[slot])

    for slot in range(RING):
        fire(slot, jnp.int32(slot))

    def wave(v, carry):
        for slot in range(RING):
            ci = v * RING + slot

            @pl.when(ci < n_my_panels)
            def _():
                c = c0 + ci
                pltpu.make_async_copy(tail_panel_hbm, panels_v.at[slot],
                                      sems[slot]).wait()

                def drain(_, carry3):
                    pltpu.make_async_copy(orows_v.at[slot],
                                          hm_hbm.at[pl.ds(0, L)],
                                          osems[slot]).wait()
                    return carry3

                lax.fori_loop(0, pend_s[slot], drain, 0)
                s = jnp.where(c > c0, start_s[jnp.maximum(ci - 1, 0)], 0)
                e = start_s[ci]

                def mchunk(i, carry2):
                    m0 = s + i * L
                    valid = (m0 + lane) < e
                    pk = bents_v[pl.ds(m0, L)]
                    pk = jnp.where(valid, pk, jnp.full((L,), pk[0], jnp.int32))
                    col = pk & 127
                    mv = (pk >> 7) & 32767
                    for d in range(DIM):
                        dsp = jnp.full((L,), d, jnp.int32)
                        vals = plsc.load_gather(panels_v.at[slot], [dsp, col])
                        plsc.store_scatter(orows_v.at[slot], [lane, dsp], vals)
                    pltpu.async_copy(orows_v.at[slot], hm_hbm.at[mv],
                                     osems[slot])
                    return carry2

                nchunks = jnp.maximum((e - s + L - 1) // L, 0)
                lax.fori_loop(0, nchunks, mchunk, 0)
                pend_s[slot] = nchunks
                fire(slot, ci + RING)
        return carry

    lax.fori_loop(0, (P_PER_W + RING - 1) // RING, wave, 0)

    for slot in range(RING):
        def fdrain(_, carry3):
            pltpu.make_async_copy(orows_v.at[slot], hm_hbm.at[pl.ds(0, L)],
                                  osems[slot]).wait()
            return carry3

        lax.fori_loop(0, pend_s[slot], fdrain, 0)


def _score_body(rel_hbm, hm_hbm, emb_rt_hbm, out_hbm,
                ridx_v, h_v, t_v, rt_v, out_v, sem):
    wid = lax.axis_index("s") * NC + lax.axis_index("c")
    base = wid * B_PER_W
    lane = lax.broadcasted_iota(jnp.int32, (L,), 0)

    for p in range(8):
        off = pl.multiple_of(p * PANEL, PANEL)
        pltpu.async_copy(emb_rt_hbm.at[:, pl.ds(off, PANEL)],
                         rt_v.at[:, pl.ds(off, PANEL)], sem)
    pltpu.make_async_copy(emb_rt_hbm, rt_v, sem).wait()

    for half in range(B_PER_W // K2CHUNK):
        b0 = base + half * K2CHUNK
        pltpu.sync_copy(rel_hbm.at[pl.ds(b0, K2CHUNK)], ridx_v)
        pltpu.async_copy(hm_hbm.at[pl.ds(b0, K2CHUNK)], h_v, sem)
        pltpu.async_copy(hm_hbm.at[pl.ds(BATCH + b0, K2CHUNK)], t_v, sem)
        pltpu.make_async_copy(hm_hbm.at[pl.ds(0, K2CHUNK)], h_v, sem).wait()
        pltpu.make_async_copy(hm_hbm.at[pl.ds(0, K2CHUNK)], t_v, sem).wait()

        def group(g, carry):
            rows = g * L + lane
            rel = ridx_v[pl.ds(g * L, L)]
            acc = jnp.zeros((L,), jnp.float32)
            for d in range(DIM):
                dsp = jnp.full((L,), d, jnp.int32)
                hh = plsc.load_gather(h_v, [rows, dsp])
                tt = plsc.load_gather(t_v, [rows, dsp])
                rr = plsc.load_gather(rt_v, [dsp, rel])
                acc = acc + hh * tt * rr
            out_v[pl.ds(half * K2CHUNK + g * L, L)] = acc
            return carry

        lax.fori_loop(0, K2CHUNK // L, group, 0)

    pltpu.sync_copy(out_v, out_hbm.at[pl.ds(base, B_PER_W)])


@jax.jit
def kernel(head, tail, relation, emb_E, emb_R):
    head = head.astype(jnp.int32)
    tail = tail.astype(jnp.int32)
    relation = relation.astype(jnp.int32)
    emb_et = emb_E.T
    tail_panel = jnp.pad(emb_E[N_PANEL_FULL * PANEL:], ((0, 64), (0, 0))).T
    emb_rt = jnp.pad(emb_R, ((0, 1024 - N_RELATION), (0, 0))).T

    sweep = pl.kernel(
        _sweep_body,
        out_type=jax.ShapeDtypeStruct((2 * BATCH, PANEL), jnp.float32),
        mesh=plsc.VectorSubcoreMesh(**_mesh),
        compiler_params=_params,
        scratch_types=[
            pltpu.VMEM((2048,), jnp.int32),
            pltpu.VMEM((MAXM,), jnp.int32),
            pltpu.VMEM((MAXM,), jnp.int32),
            pltpu.VMEM((RING, DIM, PANEL), jnp.float32),
            pltpu.VMEM((RING, L, PANEL), jnp.float32),
            pltpu.SMEM((1,), jnp.int32),
            pltpu.SMEM((P_PER_W,), jnp.int32),
            pltpu.SMEM((P_PER_W,), jnp.int32),
            pltpu.SMEM((RING,), jnp.int32),
            [pltpu.SemaphoreType.DMA] * RING,
            [pltpu.SemaphoreType.DMA] * RING,
            pltpu.SemaphoreType.DMA,
        ],
    )
    hm = sweep(head, tail, emb_et, tail_panel)

    score = pl.kernel(
        _score_body,
        out_type=jax.ShapeDtypeStruct((BATCH,), jnp.float32),
        mesh=plsc.VectorSubcoreMesh(**_mesh),
        compiler_params=_params,
        scratch_types=[
            pltpu.VMEM((K2CHUNK,), jnp.int32),
            pltpu.VMEM((K2CHUNK, PANEL), jnp.float32),
            pltpu.VMEM((K2CHUNK, PANEL), jnp.float32),
            pltpu.VMEM((DIM, 1024), jnp.float32),
            pltpu.VMEM((B_PER_W,), jnp.float32),
            pltpu.SemaphoreType.DMA,
        ],
    )
    return score(relation, hm, emb_rt)


# speedup vs baseline: 2.2354x; 1.0116x over previous
import functools

import jax
import jax.numpy as jnp
from jax import lax
from jax.experimental import pallas as pl
from jax.experimental.pallas import tpu as pltpu
from jax.experimental.pallas import tpu_sc as plsc

N_ENTITY = 1000000
N_RELATION = 1000
BATCH = 16384
DIM = 64

NC = 2
NS = 16
L = 16
NW = NC * NS
PANEL = 128
N_PANEL_FULL = N_ENTITY // PANEL
N_PANELS = N_PANEL_FULL + 1
P_PER_W = (N_PANELS + NW - 1) // NW
RING = 4
MAXM = 2 * BATCH + L
B_PER_W = BATCH // NW
K2CHUNK = 64

_params = pltpu.CompilerParams(needs_layout_passes=False, use_tc_tiling_on_sc=True)
_mesh = dict(core_axis_name="c", subcore_axis_name="s", num_cores=NC, num_subcores=NS)


def _sweep_body(head_hbm, tail_hbm, emb_et_hbm, tail_panel_hbm, hm_hbm,
                idx_v, ents_v, bents_v, panels_v, orows_v,
                cnt_s, hist_s, start_s, pend_s,
                sems, osems, sem):
    wid = lax.axis_index("s") * NC + lax.axis_index("c")
    c0 = wid * P_PER_W
    c1 = jnp.minimum(c0 + P_PER_W, N_PANELS)
    e_lo = c0 * PANEL
    e_hi = jnp.minimum(c1 * PANEL, N_ENTITY)
    lane = lax.broadcasted_iota(jnp.int32, (L,), 0)

    cnt_s[0] = 0
    for slot in range(RING):
        pend_s[slot] = 0

    def scan_block(role, src_hbm):
        def blk(i, carry):
            pltpu.sync_copy(src_hbm.at[pl.ds(i * 2048, 2048)], idx_v)

            def chunk(j, carry2):
                e = idx_v[pl.ds(j * L, L)]
                m = (e >= e_lo) & (e < e_hi)
                npop = plsc.all_reduce_population_count(m)
                cnt = cnt_s[0]
                slot = (i * 2048 + j * L + role * BATCH) + lane
                packed = (jax.lax.shift_left((e >> 7) - c0, 22)
                          | jax.lax.shift_left(slot, 7) | (e & 127))
                plsc.store_compressed(ents_v.at[pl.ds(cnt, L)], packed, mask=m)
                cnt_s[0] = cnt + npop[0]
                return carry2

            return lax.fori_loop(0, 2048 // L, chunk, carry)

        lax.fori_loop(0, BATCH // 2048, blk, 0)

    scan_block(0, head_hbm)
    scan_block(1, tail_hbm)
    nmatch = cnt_s[0]

    def hzero(p, carry):
        hist_s[p] = 0
        return carry

    lax.fori_loop(0, P_PER_W, hzero, 0)

    def hcount(i, carry):
        ev = ents_v[pl.ds(i * L, L)]
        for k in range(L):
            @pl.when((i * L + k) < nmatch)
            def _():
                p = ev[k] >> 22
                hist_s[p] = hist_s[p] + 1
        return carry

    lax.fori_loop(0, (nmatch + L - 1) // L, hcount, 0)

    def prefix(p, acc):
        h = hist_s[p]
        start_s[p] = acc
        return acc + h

    lax.fori_loop(0, P_PER_W, prefix, 0)

    def bsort(i, carry):
        ev = ents_v[pl.ds(i * L, L)]
        for k in range(L):
            @pl.when((i * L + k) < nmatch)
            def _():
                p = ev[k] >> 22
                pos = start_s[p]
                onek = lane == k
                plsc.store_compressed(bents_v.at[pl.ds(pos, L)],
                                      jnp.full((L,), ev[k], jnp.int32), mask=onek)
                start_s[p] = pos + 1
        return carry

    lax.fori_loop(0, (nmatch + L - 1) // L, bsort, 0)

    n_my_panels = c1 - c0

    def fire(slot, ci):
        @pl.when(ci < n_my_panels)
        def _():
            c = c0 + ci

            @pl.when(c < N_PANEL_FULL)
            def _():
                off = pl.multiple_of(c * PANEL, PANEL)
                pltpu.async_copy(emb_et_hbm.at[:, pl.ds(off, PANEL)],
                                 panels_v.at[slot], sems[slot])

            @pl.when(c >= N_PANEL_FULL)
            def _():
                pltpu.async_copy(tail_panel_hbm, panels_v.at[slot], sems[slot])

    for slot in range(RING):
        fire(slot, jnp.int32(slot))

    def wave(v, carry):
        for slot in range(RING):
            ci = v * RING + slot

            @pl.when(ci < n_my_panels)
            def _():
                c = c0 + ci
                pltpu.make_async_copy(tail_panel_hbm, panels_v.at[slot],
                                      sems[slot]).wait()

                def drain(_, carry3):
                    pltpu.make_async_copy(orows_v.at[slot],
                                          hm_hbm.at[pl.ds(0, L)],
                                          osems[slot]).wait()
                    return carry3

                lax.fori_loop(0, pend_s[slot], drain, 0)
                s = jnp.where(c > c0, start_s[jnp.maximum(ci - 1, 0)], 0)
                e = start_s[ci]

                def mchunk(i, carry2):
                    m0 = s + i * L
                    valid = (m0 + lane) < e
                    pk = bents_v[pl.ds(m0, L)]
                    pk = jnp.where(valid, pk, jnp.full((L,), pk[0], jnp.int32))
                    col = pk & 127
                    mv = (pk >> 7) & 32767
                    for d in range(DIM):
                        dsp = jnp.full((L,), d, jnp.int32)
                        vals = plsc.load_gather(panels_v.at[slot], [dsp, col])
                        plsc.store_scatter(orows_v.at[slot], [lane, dsp], vals)
                    pltpu.async_copy(orows_v.at[slot], hm_hbm.at[mv],
                                     osems[slot])
                    return carry2

                nchunks = jnp.maximum((e - s + L - 1) // L, 0)
                lax.fori_loop(0, nchunks, mchunk, 0)
                pend_s[slot] = nchunks
                fire(slot, ci + RING)
        return carry

    lax.fori_loop(0, (P_PER_W + RING - 1) // RING, wave, 0)

    for slot in range(RING):
        def fdrain(_, carry3):
            pltpu.make_async_copy(orows_v.at[slot], hm_hbm.at[pl.ds(0, L)],
                                  osems[slot]).wait()
            return carry3

        lax.fori_loop(0, pend_s[slot], fdrain, 0)


def _score_body(rel_hbm, hm_hbm, emb_rt_hbm, out_hbm,
                ridx_v, h_v, t_v, rt_v, out_v, sems2, sem):
    wid = lax.axis_index("s") * NC + lax.axis_index("c")
    base = wid * B_PER_W
    lane = lax.broadcasted_iota(jnp.int32, (L,), 0)
    npass = B_PER_W // K2CHUNK

    for p in range(8):
        off = pl.multiple_of(p * PANEL, PANEL)
        pltpu.async_copy(emb_rt_hbm.at[:, pl.ds(off, PANEL)],
                         rt_v.at[:, pl.ds(off, PANEL)], sem)

    def fire(slot, half):
        if half < npass:
            b0 = base + half * K2CHUNK
            pltpu.async_copy(rel_hbm.at[pl.ds(b0, K2CHUNK)],
                             ridx_v.at[pl.ds(slot * K2CHUNK, K2CHUNK)],
                             sems2[slot])
            pltpu.async_copy(hm_hbm.at[pl.ds(b0, K2CHUNK)], h_v.at[slot],
                             sems2[slot])
            pltpu.async_copy(hm_hbm.at[pl.ds(BATCH + b0, K2CHUNK)],
                             t_v.at[slot], sems2[slot])

    fire(0, 0)
    fire(1, 1)
    pltpu.make_async_copy(emb_rt_hbm, rt_v, sem).wait()

    for half in range(npass):
        slot = half % 2
        pltpu.make_async_copy(rel_hbm.at[pl.ds(0, K2CHUNK)],
                              ridx_v.at[pl.ds(slot * K2CHUNK, K2CHUNK)],
                              sems2[slot]).wait()
        pltpu.make_async_copy(hm_hbm.at[pl.ds(0, K2CHUNK)], h_v.at[slot],
                              sems2[slot]).wait()
        pltpu.make_async_copy(hm_hbm.at[pl.ds(0, K2CHUNK)], t_v.at[slot],
                              sems2[slot]).wait()

        def group(g, carry):
            rows = g * L + lane
            rel = ridx_v[pl.ds(slot * K2CHUNK + g * L, L)]
            acc = jnp.zeros((L,), jnp.float32)
            for d in range(DIM):
                dsp = jnp.full((L,), d, jnp.int32)
                hh = plsc.load_gather(h_v.at[slot], [rows, dsp])
                tt = plsc.load_gather(t_v.at[slot], [rows, dsp])
                rr = plsc.load_gather(rt_v, [dsp, rel])
                acc = acc + hh * tt * rr
            out_v[pl.ds(half * K2CHUNK + g * L, L)] = acc
            return carry

        lax.fori_loop(0, K2CHUNK // L, group, 0)
        fire(slot, half + 2)

    pltpu.sync_copy(out_v, out_hbm.at[pl.ds(base, B_PER_W)])


@jax.jit
def kernel(head, tail, relation, emb_E, emb_R):
    head = head.astype(jnp.int32)
    tail = tail.astype(jnp.int32)
    relation = relation.astype(jnp.int32)
    emb_et = emb_E.T
    tail_panel = jnp.pad(emb_E[N_PANEL_FULL * PANEL:], ((0, 64), (0, 0))).T
    emb_rt = jnp.pad(emb_R, ((0, 1024 - N_RELATION), (0, 0))).T

    sweep = pl.kernel(
        _sweep_body,
        out_type=jax.ShapeDtypeStruct((2 * BATCH, PANEL), jnp.float32),
        mesh=plsc.VectorSubcoreMesh(**_mesh),
        compiler_params=_params,
        scratch_types=[
            pltpu.VMEM((2048,), jnp.int32),
            pltpu.VMEM((MAXM,), jnp.int32),
            pltpu.VMEM((MAXM,), jnp.int32),
            pltpu.VMEM((RING, DIM, PANEL), jnp.float32),
            pltpu.VMEM((RING, L, PANEL), jnp.float32),
            pltpu.SMEM((1,), jnp.int32),
            pltpu.SMEM((P_PER_W,), jnp.int32),
            pltpu.SMEM((P_PER_W,), jnp.int32),
            pltpu.SMEM((RING,), jnp.int32),
            [pltpu.SemaphoreType.DMA] * RING,
            [pltpu.SemaphoreType.DMA] * RING,
            pltpu.SemaphoreType.DMA,
        ],
    )
    hm = sweep(head, tail, emb_et, tail_panel)

    score = pl.kernel(
        _score_body,
        out_type=jax.ShapeDtypeStruct((BATCH,), jnp.float32),
        mesh=plsc.VectorSubcoreMesh(**_mesh),
        compiler_params=_params,
        scratch_types=[
            pltpu.VMEM((2 * K2CHUNK,), jnp.int32),
            pltpu.VMEM((2, K2CHUNK, PANEL), jnp.float32),
            pltpu.VMEM((2, K2CHUNK, PANEL), jnp.float32),
            pltpu.VMEM((DIM, 1024), jnp.float32),
            pltpu.VMEM((B_PER_W,), jnp.float32),
            [pltpu.SemaphoreType.DMA] * 2,
            pltpu.SemaphoreType.DMA,
        ],
    )
    return score(relation, hm, emb_rt)


# speedup vs baseline: 2.7316x; 1.2220x over previous
import functools

import jax
import jax.numpy as jnp
from jax import lax
from jax.experimental import pallas as pl
from jax.experimental.pallas import tpu as pltpu
from jax.experimental.pallas import tpu_sc as plsc

N_ENTITY = 1000000
N_RELATION = 1000
BATCH = 16384
DIM = 64

NC = 2
NS = 16
L = 16
NW = NC * NS
PANEL = 128
N_PANEL_FULL = N_ENTITY // PANEL
N_PANELS = N_PANEL_FULL + 1
P_PER_W = (N_PANELS + NW - 1) // NW
RING = 4
MAXM = 2 * BATCH + L
B_PER_W = BATCH // NW
K2CHUNK = 64

_params = pltpu.CompilerParams(needs_layout_passes=False, use_tc_tiling_on_sc=True)
_mesh = dict(core_axis_name="c", subcore_axis_name="s", num_cores=NC, num_subcores=NS)


def _sweep_body(head_hbm, tail_hbm, emb_et_hbm, tail_panel_hbm, hm_hbm,
                idx_v, ents_v, bents_v, panels_v, orows_v,
                cnt_s, hist_s, start_s, pend_s,
                sems, osems, sem):
    wid = lax.axis_index("s") * NC + lax.axis_index("c")
    c0 = wid * P_PER_W
    c1 = jnp.minimum(c0 + P_PER_W, N_PANELS)
    e_lo = c0 * PANEL
    e_hi = jnp.minimum(c1 * PANEL, N_ENTITY)
    lane = lax.broadcasted_iota(jnp.int32, (L,), 0)

    cnt_s[0] = 0
    for slot in range(RING):
        pend_s[slot] = 0

    def scan_block(role, src_hbm):
        def blk(i, carry):
            pltpu.sync_copy(src_hbm.at[pl.ds(i * 2048, 2048)], idx_v)

            def chunk(j, carry2):
                e = idx_v[pl.ds(j * L, L)]
                m = (e >= e_lo) & (e < e_hi)
                npop = plsc.all_reduce_population_count(m)
                cnt = cnt_s[0]
                slot = (i * 2048 + j * L + role * BATCH) + lane
                packed = (jax.lax.shift_left((e >> 7) - c0, 22)
                          | jax.lax.shift_left(slot, 7) | (e & 127))
                plsc.store_compressed(ents_v.at[pl.ds(cnt, L)], packed, mask=m)
                cnt_s[0] = cnt + npop[0]
                return carry2

            return lax.fori_loop(0, 2048 // L, chunk, carry)

        lax.fori_loop(0, BATCH // 2048, blk, 0)

    scan_block(0, head_hbm)
    scan_block(1, tail_hbm)
    nmatch = cnt_s[0]

    def hzero(p, carry):
        hist_s[p] = 0
        return carry

    lax.fori_loop(0, P_PER_W, hzero, 0)

    def hcount(i, carry):
        ev = ents_v[pl.ds(i * L, L)]
        for k in range(L):
            @pl.when((i * L + k) < nmatch)
            def _():
                p = ev[k] >> 22
                hist_s[p] = hist_s[p] + 1
        return carry

    lax.fori_loop(0, (nmatch + L - 1) // L, hcount, 0)

    def prefix(p, acc):
        h = hist_s[p]
        start_s[p] = acc
        return acc + h

    lax.fori_loop(0, P_PER_W, prefix, 0)

    def bsort(i, carry):
        ev = ents_v[pl.ds(i * L, L)]
        for k in range(L):
            @pl.when((i * L + k) < nmatch)
            def _():
                p = ev[k] >> 22
                pos = start_s[p]
                onek = lane == k
                plsc.store_compressed(bents_v.at[pl.ds(pos, L)],
                                      jnp.full((L,), ev[k], jnp.int32), mask=onek)
                start_s[p] = pos + 1
        return carry

    lax.fori_loop(0, (nmatch + L - 1) // L, bsort, 0)

    n_my_panels = c1 - c0

    def fire(slot, ci):
        @pl.when(ci < n_my_panels)
        def _():
            c = c0 + ci

            @pl.when(c < N_PANEL_FULL)
            def _():
                off = pl.multiple_of(c * PANEL, PANEL)
                pltpu.async_copy(emb_et_hbm.at[:, pl.ds(off, PANEL)],
                                 panels_v.at[slot], sems[slot])

            @pl.when(c >= N_PANEL_FULL)
            def _():
                pltpu.async_copy(tail_panel_hbm, panels_v.at[slot], sems[slot])

    for slot in range(RING):
        fire(slot, jnp.int32(slot))

    def wave(v, carry):
        for slot in range(RING):
            ci = v * RING + slot

            @pl.when(ci < n_my_panels)
            def _():
                c = c0 + ci
                pltpu.make_async_copy(tail_panel_hbm, panels_v.at[slot],
                                      sems[slot]).wait()

                def drain(_, carry3):
                    pltpu.make_async_copy(orows_v.at[slot],
                                          hm_hbm.at[pl.ds(0, L)],
                                          osems[slot]).wait()
                    return carry3

                lax.fori_loop(0, pend_s[slot], drain, 0)
                s = jnp.where(c > c0, start_s[jnp.maximum(ci - 1, 0)], 0)
                e = start_s[ci]

                def mchunk(i, carry2):
                    m0 = s + i * L
                    valid = (m0 + lane) < e
                    pk = bents_v[pl.ds(m0, L)]
                    pk = jnp.where(valid, pk, jnp.full((L,), pk[0], jnp.int32))
                    col = pk & 127
                    mv = (pk >> 7) & 32767
                    smod = mv & 63
                    for d in range(DIM):
                        dsp = jnp.full((L,), d, jnp.int32)
                        vals = plsc.load_gather(panels_v.at[slot], [dsp, col])
                        plsc.store_scatter(orows_v.at[slot],
                                           [lane, (dsp + smod) & 63], vals)
                    pltpu.async_copy(orows_v.at[slot], hm_hbm.at[mv],
                                     osems[slot])
                    return carry2

                nchunks = jnp.maximum((e - s + L - 1) // L, 0)
                lax.fori_loop(0, nchunks, mchunk, 0)
                pend_s[slot] = nchunks
                fire(slot, ci + RING)
        return carry

    lax.fori_loop(0, (P_PER_W + RING - 1) // RING, wave, 0)

    for slot in range(RING):
        def fdrain(_, carry3):
            pltpu.make_async_copy(orows_v.at[slot], hm_hbm.at[pl.ds(0, L)],
                                  osems[slot]).wait()
            return carry3

        lax.fori_loop(0, pend_s[slot], fdrain, 0)


def _score_body(rel_hbm, hm_hbm, emb_rt_hbm, out_hbm,
                ridx_v, h_v, t_v, rt_v, out_v, sems2, sem):
    wid = lax.axis_index("s") * NC + lax.axis_index("c")
    base = wid * B_PER_W
    lane = lax.broadcasted_iota(jnp.int32, (L,), 0)
    npass = B_PER_W // K2CHUNK

    for p in range(8):
        off = pl.multiple_of(p * PANEL, PANEL)
        pltpu.async_copy(emb_rt_hbm.at[:, pl.ds(off, PANEL)],
                         rt_v.at[:, pl.ds(off, PANEL)], sem)

    def fire(slot, half):
        if half < npass:
            b0 = base + half * K2CHUNK
            pltpu.async_copy(rel_hbm.at[pl.ds(b0, K2CHUNK)],
                             ridx_v.at[pl.ds(slot * K2CHUNK, K2CHUNK)],
                             sems2[slot])
            pltpu.async_copy(hm_hbm.at[pl.ds(b0, K2CHUNK)], h_v.at[slot],
                             sems2[slot])
            pltpu.async_copy(hm_hbm.at[pl.ds(BATCH + b0, K2CHUNK)],
                             t_v.at[slot], sems2[slot])

    fire(0, 0)
    fire(1, 1)
    pltpu.make_async_copy(emb_rt_hbm, rt_v, sem).wait()

    for half in range(npass):
        slot = half % 2
        b0 = base + half * K2CHUNK
        pltpu.make_async_copy(rel_hbm.at[pl.ds(0, K2CHUNK)],
                              ridx_v.at[pl.ds(slot * K2CHUNK, K2CHUNK)],
                              sems2[slot]).wait()
        pltpu.make_async_copy(hm_hbm.at[pl.ds(0, K2CHUNK)], h_v.at[slot],
                              sems2[slot]).wait()
        pltpu.make_async_copy(hm_hbm.at[pl.ds(0, K2CHUNK)], t_v.at[slot],
                              sems2[slot]).wait()

        def group(g, carry):
            rows = g * L + lane
            rel = ridx_v[pl.ds(slot * K2CHUNK + g * L, L)]
            smod = (b0 + g * L + lane) & 63
            acc = jnp.zeros((L,), jnp.float32)
            for d in range(DIM):
                dsp = jnp.full((L,), d, jnp.int32)
                sk = (dsp + smod) & 63
                hh = plsc.load_gather(h_v.at[slot], [rows, sk])
                tt = plsc.load_gather(t_v.at[slot], [rows, sk])
                rr = plsc.load_gather(rt_v, [dsp, rel])
                acc = acc + hh * tt * rr
            out_v[pl.ds(half * K2CHUNK + g * L, L)] = acc
            return carry

        lax.fori_loop(0, K2CHUNK // L, group, 0)
        fire(slot, half + 2)

    pltpu.sync_copy(out_v, out_hbm.at[pl.ds(base, B_PER_W)])


@jax.jit
def kernel(head, tail, relation, emb_E, emb_R):
    head = head.astype(jnp.int32)
    tail = tail.astype(jnp.int32)
    relation = relation.astype(jnp.int32)
    emb_et = emb_E.T
    tail_panel = jnp.pad(emb_E[N_PANEL_FULL * PANEL:], ((0, 64), (0, 0))).T
    emb_rt = jnp.pad(emb_R, ((0, 1024 - N_RELATION), (0, 0))).T

    sweep = pl.kernel(
        _sweep_body,
        out_type=jax.ShapeDtypeStruct((2 * BATCH, PANEL), jnp.float32),
        mesh=plsc.VectorSubcoreMesh(**_mesh),
        compiler_params=_params,
        scratch_types=[
            pltpu.VMEM((2048,), jnp.int32),
            pltpu.VMEM((MAXM,), jnp.int32),
            pltpu.VMEM((MAXM,), jnp.int32),
            pltpu.VMEM((RING, DIM, PANEL), jnp.float32),
            pltpu.VMEM((RING, L, PANEL), jnp.float32),
            pltpu.SMEM((1,), jnp.int32),
            pltpu.SMEM((P_PER_W,), jnp.int32),
            pltpu.SMEM((P_PER_W,), jnp.int32),
            pltpu.SMEM((RING,), jnp.int32),
            [pltpu.SemaphoreType.DMA] * RING,
            [pltpu.SemaphoreType.DMA] * RING,
            pltpu.SemaphoreType.DMA,
        ],
    )
    hm = sweep(head, tail, emb_et, tail_panel)

    score = pl.kernel(
        _score_body,
        out_type=jax.ShapeDtypeStruct((BATCH,), jnp.float32),
        mesh=plsc.VectorSubcoreMesh(**_mesh),
        compiler_params=_params,
        scratch_types=[
            pltpu.VMEM((2 * K2CHUNK,), jnp.int32),
            pltpu.VMEM((2, K2CHUNK, PANEL), jnp.float32),
            pltpu.VMEM((2, K2CHUNK, PANEL), jnp.float32),
            pltpu.VMEM((DIM, 1024), jnp.float32),
            pltpu.VMEM((B_PER_W,), jnp.float32),
            [pltpu.SemaphoreType.DMA] * 2,
            pltpu.SemaphoreType.DMA,
        ],
    )
    return score(relation, hm, emb_rt)
